# baseline (speedup 1.0000x reference)
import functools
import jax
import jax.numpy as jnp
from jax import lax
from jax.experimental import pallas as pl
from jax.experimental.pallas import tpu as pltpu
from jax.experimental.pallas import tpu_sc as plsc

CH = 128


def _tc_body(x0, x1, x2, x3, o_ref):
    o_ref[0, 0, :] = x0[0, 0, :] + x1[0, 0, :] + x2[0, 0, :] + x3[0, 0, :]


def _sc_body(x_hbm, out_hbm, buf, sem):
    c = lax.axis_index("c")
    s = lax.axis_index("s")
    wid = s * 2 + c
    b = wid // 2
    h = wid % 2
    base = b * 4096 + 3072 + h * 512
    nch = 512 // CH

    def body(i, _):
        pltpu.async_copy(x_hbm.at[pl.ds(base + i * CH, CH)], buf, sem).wait()
        return 0

    lax.fori_loop(0, nch, body, 0)
    pltpu.sync_copy(buf.at[0, pl.ds(0, 16)], out_hbm.at[wid])


def kernel(inputs):
    B, S, D = inputs.shape
    Q = 3072 // 4
    def spec(q):
        return pl.BlockSpec((1, Q, D), lambda b, q=q: (b, q, 0))
    tc_out = pl.pallas_call(
        _tc_body,
        grid=(B,),
        in_specs=[spec(q) for q in range(4)],
        out_specs=pl.BlockSpec((1, 1, D), lambda b: (b, 0, 0)),
        out_shape=jax.ShapeDtypeStruct((B, 1, D), inputs.dtype),
    )(*([inputs] * 4))

    flat = inputs.reshape(B * S, D)
    mesh = plsc.VectorSubcoreMesh(core_axis_name="c", subcore_axis_name="s")
    sc_out = functools.partial(
        pl.kernel,
        out_type=jax.ShapeDtypeStruct((32, 16), jnp.float32),
        mesh=mesh,
        scratch_types=[
            pltpu.VMEM((CH, D), jnp.float32),
            pltpu.SemaphoreType.DMA,
        ],
    )(_sc_body)(flat)

    return tc_out.reshape(B, D) + jnp.sum(sc_out)


# speedup vs baseline: 1.2936x; 1.2936x over previous
import jax
import jax.numpy as jnp
from jax.experimental import pallas as pl

NSTREAM = 4


def _count(x):
    m = jnp.max(jnp.abs(x), axis=1)
    return jnp.sum((m > 0).astype(jnp.int32))


def _body(x0, x1, x2, x3, o_ref):
    refs = (x0, x1, x2, x3)
    sq = x0.shape[1]
    cnt = _count(x0[0]) + _count(x1[0]) + _count(x2[0]) + _count(x3[0])
    idx = jnp.maximum(cnt - 1, 0)
    q = idx // sq
    off = idx % sq
    row = refs[NSTREAM - 1][0, pl.ds(off, 1), :]
    for i in range(NSTREAM - 2, -1, -1):
        row = jnp.where(q == i, refs[i][0, pl.ds(off, 1), :], row)
    o_ref[0, :, :] = row


def kernel(inputs):
    B, S, D = inputs.shape
    Q = S // NSTREAM

    def spec(q):
        return pl.BlockSpec((1, Q, D), lambda b, q=q: (b, q, 0))

    out = pl.pallas_call(
        _body,
        grid=(B,),
        in_specs=[spec(q) for q in range(NSTREAM)],
        out_specs=pl.BlockSpec((1, 1, D), lambda b: (b, 0, 0)),
        out_shape=jax.ShapeDtypeStruct((B, 1, D), inputs.dtype),
    )(*([inputs] * NSTREAM))
    return out.reshape(B, D)


# speedup vs baseline: 1.2949x; 1.0010x over previous
import jax
import jax.numpy as jnp
from jax.experimental import pallas as pl

NSTREAM = 4


def _count(x, ones_j):
    sq, d = x.shape
    chunks = [jnp.abs(x[:, c * 128:(c + 1) * 128]) for c in range(d // 128)]
    while len(chunks) > 1:
        chunks = [
            jnp.maximum(chunks[i], chunks[i + 1])
            if i + 1 < len(chunks) else chunks[i]
            for i in range(0, len(chunks), 2)
        ]
    rs = jax.lax.dot_general(
        chunks[0].astype(jnp.bfloat16), ones_j, (((1,), (0,)), ((), ())),
        preferred_element_type=jnp.float32,
    )
    return jnp.sum(jnp.minimum(rs, 1.0))


def _body(x0, x1, x2, x3, o_ref):
    refs = (x0, x1, x2, x3)
    sq = x0.shape[1]
    ones_j = jnp.ones((128, 128), dtype=jnp.bfloat16)
    cnt_f = (
        _count(x0[0], ones_j) + _count(x1[0], ones_j)
        + _count(x2[0], ones_j) + _count(x3[0], ones_j)
    ) * (1.0 / 128.0)
    idx = jnp.maximum(cnt_f - 1.0, 0.0).astype(jnp.int32)
    q = idx // sq
    off = idx % sq
    row = refs[NSTREAM - 1][0, pl.ds(off, 1), :]
    for i in range(NSTREAM - 2, -1, -1):
        row = jnp.where(q == i, refs[i][0, pl.ds(off, 1), :], row)
    o_ref[0, :, :] = row


def kernel(inputs):
    B, S, D = inputs.shape
    Q = S // NSTREAM

    def spec(q):
        return pl.BlockSpec((1, Q, D), lambda b, q=q: (b, q, 0))

    out = pl.pallas_call(
        _body,
        grid=(B,),
        in_specs=[spec(q) for q in range(NSTREAM)],
        out_specs=pl.BlockSpec((1, 1, D), lambda b: (b, 0, 0)),
        out_shape=jax.ShapeDtypeStruct((B, 1, D), inputs.dtype),
    )(*([inputs] * NSTREAM))
    return out.reshape(B, D)


# speedup vs baseline: 1.3272x; 1.0250x over previous
import jax
import jax.numpy as jnp
from jax.experimental import pallas as pl

NSTREAM = 4


def _count(x, ones_j):
    sq, d = x.shape
    chunks = [jnp.abs(x[:, c * 128:(c + 1) * 128]) for c in range(d // 128)]
    while len(chunks) > 1:
        chunks = [
            jnp.maximum(chunks[i], chunks[i + 1])
            if i + 1 < len(chunks) else chunks[i]
            for i in range(0, len(chunks), 2)
        ]
    rs = jax.lax.dot_general(
        chunks[0].astype(jnp.bfloat16), ones_j, (((1,), (0,)), ((), ())),
        preferred_element_type=jnp.float32,
    )
    return jnp.sum(jnp.minimum(rs, 1.0))


def _body(x0, x1, x2, x3, o_ref):
    refs = (x0, x1, x2, x3)
    sq = x0.shape[1]
    ones_j = jnp.ones((128, 128), dtype=jnp.bfloat16)
    cnt_f = (
        _count(x0[0], ones_j) + _count(x1[0], ones_j)
        + _count(x2[0], ones_j) + _count(x3[0], ones_j)
    ) * (1.0 / 128.0)
    idx = jnp.maximum(cnt_f - 1.0, 0.0).astype(jnp.int32)
    q = idx // sq
    off = idx % sq
    row = refs[NSTREAM - 1][0, pl.ds(off, 1), :]
    for i in range(NSTREAM - 2, -1, -1):
        row = jnp.where(q == i, refs[i][0, pl.ds(off, 1), :], row)
    b = pl.program_id(0)
    o_ref[pl.ds(b, 1), :] = row


def kernel(inputs):
    B, S, D = inputs.shape
    Q = S // NSTREAM

    def spec(q):
        return pl.BlockSpec((1, Q, D), lambda b, q=q: (b, q, 0))

    return pl.pallas_call(
        _body,
        grid=(B,),
        in_specs=[spec(q) for q in range(NSTREAM)],
        out_specs=pl.BlockSpec((B, D), lambda b: (0, 0)),
        out_shape=jax.ShapeDtypeStruct((B, D), inputs.dtype),
    )(*([inputs] * NSTREAM))
